# Initial kernel scaffold; baseline (speedup 1.0000x reference)
import jax
import jax.numpy as jnp
from jax.experimental import pallas as pl


def kernel(x, edge_index, edge_attr, node_type, edge_type, params):
    raise NotImplementedError("write your pallas kernel here")


# speedup vs baseline: 1.0338x; 1.0338x over previous
import functools

import jax
import jax.numpy as jnp
from jax.experimental import pallas as pl
from jax.experimental.pallas import tpu as pltpu

H = 64
HEADS = 4
HD = 16
L = 2


def _heads_body(we_ref, r1w, r1b, r2w, r2b, r3w, r3b, c1w, c1b, c2w, c2b,
                e1w, e1b, e2w, e2b, out_ref):
    we = we_ref[...]
    r = jnp.maximum(we @ r1w[...] + r1b[...], 0.0)
    r = jnp.maximum(r @ r2w[...] + r2b[...], 0.0)
    res = jax.nn.sigmoid(r @ r3w[...] + r3b[...])
    c = jnp.maximum(we @ c1w[...] + c1b[...], 0.0)
    conf = jax.nn.sigmoid(c @ c2w[...] + c2b[...])
    e = jnp.maximum(we @ e1w[...] + e1b[...], 0.0)
    z = e @ e2w[...] + e2b[...]
    z = z - jnp.max(z, axis=-1, keepdims=True)
    pz = jnp.exp(z)
    pz = pz / jnp.sum(pz, axis=-1, keepdims=True)
    out_ref[:, 0:1] = res
    out_ref[:, 1:2] = conf
    out_ref[:, 2:7] = pz
    out_ref[:, 7:8] = res


def _heads(we, p):
    n = we.shape[0]
    bn = 2000
    grid = (n // bn,)
    full = lambda s: pl.BlockSpec(s, lambda i: (0, 0))
    args = [
        p['r1_W'], p['r1_b'].reshape(1, -1),
        p['r2_W'], p['r2_b'].reshape(1, -1),
        p['r3_W'], p['r3_b'].reshape(1, -1),
        p['c1_W'], p['c1_b'].reshape(1, -1),
        p['c2_W'], p['c2_b'].reshape(1, -1),
        p['e1_W'], p['e1_b'].reshape(1, -1),
        p['e2_W'], p['e2_b'].reshape(1, -1),
    ]
    in_specs = [pl.BlockSpec((bn, H), lambda i: (i, 0))]
    in_specs += [full(a.shape) for a in args]
    out = pl.pallas_call(
        _heads_body,
        grid=grid,
        in_specs=in_specs,
        out_specs=pl.BlockSpec((bn, 8), lambda i: (i, 0)),
        out_shape=jax.ShapeDtypeStruct((n, 8), jnp.float32),
    )(we, *args)
    return out[:, 0], out[:, 1], out[:, 2:7]


def kernel(x, edge_index, edge_attr, node_type, edge_type, params):
    p = params
    n = x.shape[0]
    e = edge_index.shape[1]
    src, dst = edge_index[0], edge_index[1]
    worker = node_type == 0
    h = jnp.where(worker[:, None], x @ p['wk_W'] + p['wk_b'], 0.0)
    deg = jnp.clip(jax.ops.segment_sum(jnp.ones((e,), jnp.float32), dst,
                                       num_segments=n), 1.0, None)
    for i in range(L):
        agg = jax.ops.segment_sum(h[src], dst, num_segments=n) / deg[:, None]
        sage = jax.nn.relu(agg @ p[f'sage_l_W_{i}'] + p[f'sage_l_b_{i}']
                           + h @ p[f'sage_r_W_{i}'])
        g = (h @ p[f'gat_W_{i}']).reshape(n, HEADS, HD)
        a_src = (g * p[f'gat_as_{i}'][None]).sum(-1)
        a_dst = (g * p[f'gat_ad_{i}'][None]).sum(-1)
        alpha = jax.nn.leaky_relu(a_src[src] + a_dst[dst], negative_slope=0.2)
        ex = jnp.exp(alpha)
        denom = jax.ops.segment_sum(ex, dst, num_segments=n)
        w = ex / jnp.clip(denom[dst], 1e-16, None)
        att = jax.ops.segment_sum(g[src] * w[:, :, None], dst,
                                  num_segments=n).reshape(n, H) + p[f'gat_b_{i}']
        h = sage + att + h
        mu = h.mean(-1, keepdims=True)
        var = ((h - mu) ** 2).mean(-1, keepdims=True)
        h = (h - mu) / jnp.sqrt(var + 1e-5)
    we = h
    resilience, confidence, explanation = _heads(we, p)
    return resilience, confidence, explanation, we


# speedup vs baseline: 35.0049x; 33.8610x over previous
import functools

import jax
import jax.numpy as jnp
from jax import lax
from jax.experimental import pallas as pl
from jax.experimental.pallas import tpu as pltpu
from jax.experimental.pallas import tpu_sc as plsc

H = 64
HEADS = 4
HD = 16
L = 2
NCORES = 2
NTILES = 16
BN = 2000


def _mesh():
    return plsc.VectorSubcoreMesh(core_axis_name="c", subcore_axis_name="s")


def _sc_agg(h_lo, h_hi, src, dst):
    n = h_lo.shape[0]
    e = src.shape[0]
    pte = e // NTILES
    K = 200
    nwin = pte // K
    ZR = 112
    npad = -(-n // (NTILES * ZR)) * (NTILES * ZR)
    ptn = npad // NTILES
    nz = ptn // ZR

    @functools.partial(
        pl.kernel, mesh=_mesh(),
        compiler_params=pltpu.CompilerParams(use_tc_tiling_on_sc=False),
        out_type=jax.ShapeDtypeStruct((NCORES, npad, 32), jnp.float32),
        scratch_types=[
            pltpu.VMEM((K,), jnp.int32),
            pltpu.VMEM((K,), jnp.int32),
            pltpu.VMEM((K, 32), jnp.float32),
            pltpu.VMEM((ZR, 32), jnp.float32),
            pltpu.VMEM_SHARED((npad, 32), jnp.float32),
            pltpu.SemaphoreType.DMA,
        ],
    )
    def k(hlo_h, hhi_h, src_h, dst_h, out_h, sidx, didx, rows, zbuf, acc, sem):
        core = lax.axis_index("c")
        sid = lax.axis_index("s")

        def zb(i, carry):
            zbuf[i, pl.ds(0, 16)] = jnp.zeros((16,), jnp.float32)
            zbuf[i, pl.ds(16, 16)] = jnp.zeros((16,), jnp.float32)
            return carry

        lax.fori_loop(0, ZR, zb, 0)
        r0 = sid * ptn
        for z in range(nz):
            pltpu.sync_copy(zbuf, acc.at[pl.ds(r0 + z * ZR, ZR)])
        plsc.subcore_barrier()
        base = sid * pte

        def win(w, carry):
            b = base + w * K
            pltpu.sync_copy(src_h.at[pl.ds(b, K)], sidx)
            pltpu.sync_copy(dst_h.at[pl.ds(b, K)], didx)

            @pl.when(core == 0)
            def _():
                pltpu.async_copy(hlo_h.at[sidx], rows, sem).wait()

            @pl.when(core == 1)
            def _():
                pltpu.async_copy(hhi_h.at[sidx], rows, sem).wait()

            pltpu.sync_copy(rows, acc.at[didx], add=True)
            return carry

        lax.fori_loop(0, nwin, win, 0)
        plsc.subcore_barrier()
        for z in range(nz):
            rr = r0 + z * ZR
            pltpu.sync_copy(acc.at[pl.ds(rr, ZR)], rows.at[pl.ds(0, ZR)])
            pltpu.sync_copy(rows.at[pl.ds(0, ZR)], out_h.at[core, pl.ds(rr, ZR)])

    return k(h_lo, h_hi, src, dst)


def _sc_attstats(as16, ad16, src, dst):
    n = as16.shape[0]
    e = src.shape[0]
    he = e // NCORES
    pte = he // NTILES
    K = 200
    nwin = pte // K
    ZR = 112
    npad = -(-n // (NTILES * ZR)) * (NTILES * ZR)
    ptn = npad // NTILES
    nz = ptn // ZR

    @functools.partial(
        pl.kernel, mesh=_mesh(),
        compiler_params=pltpu.CompilerParams(use_tc_tiling_on_sc=False),
        out_type=[
            jax.ShapeDtypeStruct((e, 16), jnp.float32),
            jax.ShapeDtypeStruct((NCORES, npad, 16), jnp.float32),
        ],
        scratch_types=[
            pltpu.VMEM((K,), jnp.int32),
            pltpu.VMEM((K,), jnp.int32),
            pltpu.VMEM((K, 16), jnp.float32),
            pltpu.VMEM((K, 16), jnp.float32),
            pltpu.VMEM((K, 16), jnp.float32),
            pltpu.VMEM((ZR, 16), jnp.float32),
            pltpu.VMEM_SHARED((npad, 16), jnp.float32),
            pltpu.SemaphoreType.DMA,
        ],
    )
    def k(as_h, ad_h, src_h, dst_h, ex_h, ddp_h,
          sidx, didx, asb, adb, exb, zbuf, acc, sem):
        core = lax.axis_index("c")
        sid = lax.axis_index("s")

        def zb(i, carry):
            zbuf[i, pl.ds(0, 16)] = jnp.zeros((16,), jnp.float32)
            return carry

        lax.fori_loop(0, ZR, zb, 0)
        r0 = sid * ptn
        for z in range(nz):
            pltpu.sync_copy(zbuf, acc.at[pl.ds(r0 + z * ZR, ZR)])
        plsc.subcore_barrier()
        base = core * he + sid * pte

        def win(w, carry):
            b = base + w * K
            pltpu.sync_copy(src_h.at[pl.ds(b, K)], sidx)
            pltpu.sync_copy(dst_h.at[pl.ds(b, K)], didx)
            pltpu.async_copy(as_h.at[sidx], asb, sem).wait()
            pltpu.async_copy(ad_h.at[didx], adb, sem).wait()

            def body(j, carry2):
                xv = asb[j, :] + adb[j, :]
                exb[j, :] = jnp.exp(jnp.maximum(xv, 0.2 * xv))
                return carry2

            lax.fori_loop(0, K, body, 0)
            pltpu.sync_copy(exb, acc.at[didx], add=True)
            pltpu.sync_copy(exb, ex_h.at[pl.ds(b, K)])
            return carry

        lax.fori_loop(0, nwin, win, 0)
        plsc.subcore_barrier()
        for z in range(nz):
            rr = r0 + z * ZR
            pltpu.sync_copy(acc.at[pl.ds(rr, ZR)], asb.at[pl.ds(0, ZR)])
            pltpu.sync_copy(asb.at[pl.ds(0, ZR)], ddp_h.at[core, pl.ds(rr, ZR)])

    return k(as16, ad16, src, dst)


_GD = lax.GatherDimensionNumbers(offset_dims=(), collapsed_slice_dims=(0,),
                                 start_index_map=(0,))


def _lane_gather(v, idx):
    return lax.gather(v, idx[:, None], _GD, (1,),
                      mode=lax.GatherScatterMode.PROMISE_IN_BOUNDS)


def _sc_att(g_lo, g_hi, dinv16, ex16, src, dst):
    n = g_lo.shape[0]
    e = src.shape[0]
    pte = e // NTILES
    K = 200
    nwin = pte // K
    ZR = 112
    npad = -(-n // (NTILES * ZR)) * (NTILES * ZR)
    ptn = npad // NTILES
    nz = ptn // ZR

    @functools.partial(
        pl.kernel, mesh=_mesh(),
        compiler_params=pltpu.CompilerParams(use_tc_tiling_on_sc=False),
        out_type=jax.ShapeDtypeStruct((NCORES, npad, 32), jnp.float32),
        scratch_types=[
            pltpu.VMEM((K,), jnp.int32),
            pltpu.VMEM((K,), jnp.int32),
            pltpu.VMEM((K, 32), jnp.float32),
            pltpu.VMEM((K, 16), jnp.float32),
            pltpu.VMEM((K, 16), jnp.float32),
            pltpu.VMEM((ZR, 32), jnp.float32),
            pltpu.VMEM_SHARED((npad, 32), jnp.float32),
            pltpu.SemaphoreType.DMA,
        ],
    )
    def k(glo_h, ghi_h, dinv_h, ex_h, src_h, dst_h, out_h,
          sidx, didx, grow, dvb, exb, zbuf, acc, sem):
        core = lax.axis_index("c")
        sid = lax.axis_index("s")
        i0 = jnp.full((16,), core * 2, jnp.int32)
        i1 = jnp.full((16,), core * 2 + 1, jnp.int32)

        def zb(i, carry):
            zbuf[i, pl.ds(0, 16)] = jnp.zeros((16,), jnp.float32)
            zbuf[i, pl.ds(16, 16)] = jnp.zeros((16,), jnp.float32)
            return carry

        lax.fori_loop(0, ZR, zb, 0)
        r0 = sid * ptn
        for z in range(nz):
            pltpu.sync_copy(zbuf, acc.at[pl.ds(r0 + z * ZR, ZR)])
        plsc.subcore_barrier()
        base = sid * pte

        def win(w, carry):
            b = base + w * K
            pltpu.sync_copy(src_h.at[pl.ds(b, K)], sidx)
            pltpu.sync_copy(dst_h.at[pl.ds(b, K)], didx)

            @pl.when(core == 0)
            def _():
                pltpu.async_copy(glo_h.at[sidx], grow, sem).wait()

            @pl.when(core == 1)
            def _():
                pltpu.async_copy(ghi_h.at[sidx], grow, sem).wait()

            pltpu.async_copy(dinv_h.at[didx], dvb, sem).wait()
            pltpu.sync_copy(ex_h.at[pl.ds(b, K)], exb)

            def body(j, carry2):
                wv = exb[j, :] * dvb[j, :]
                w0 = _lane_gather(wv, i0)
                w1 = _lane_gather(wv, i1)
                grow[j, pl.ds(0, 16)] = grow[j, pl.ds(0, 16)] * w0
                grow[j, pl.ds(16, 16)] = grow[j, pl.ds(16, 16)] * w1
                return carry2

            lax.fori_loop(0, K, body, 0)
            pltpu.sync_copy(grow, acc.at[didx], add=True)
            return carry

        lax.fori_loop(0, nwin, win, 0)
        plsc.subcore_barrier()
        for z in range(nz):
            rr = r0 + z * ZR
            pltpu.sync_copy(acc.at[pl.ds(rr, ZR)], grow.at[pl.ds(0, ZR)])
            pltpu.sync_copy(grow.at[pl.ds(0, ZR)], out_h.at[core, pl.ds(rr, ZR)])

    return k(g_lo, g_hi, dinv16, ex16, src, dst)


def _full(a):
    return pl.BlockSpec(a.shape, lambda i: tuple(0 for _ in a.shape))


def _row(width):
    return pl.BlockSpec((BN, width), lambda i: (i, 0))


def _prep_body(x_ref, nt_ref, wkw, wkb, gw, As, Ad,
               h_ref, hlo_ref, hhi_ref, glo_ref, ghi_ref, as_ref, ad_ref):
    m = (nt_ref[...] == 0).astype(jnp.float32)
    h = (x_ref[...] @ wkw[...] + wkb[...]) * m
    h_ref[...] = h
    hlo_ref[...] = h[:, :32]
    hhi_ref[...] = h[:, 32:]
    g = h @ gw[...]
    glo_ref[...] = g[:, :32]
    ghi_ref[...] = g[:, 32:]
    zs = jnp.zeros((h.shape[0], 12), jnp.float32)
    as_ref[...] = jnp.concatenate([g @ As[...], zs], axis=1)
    ad_ref[...] = jnp.concatenate([g @ Ad[...], zs], axis=1)


def _tc_prep(x, nt2, wkw, wkb, gw, As, Ad):
    n = x.shape[0]
    f = jax.ShapeDtypeStruct
    args = (x, nt2, wkw, wkb, gw, As, Ad)
    return pl.pallas_call(
        _prep_body,
        grid=(n // BN,),
        in_specs=[_row(x.shape[1]), _row(1), _full(wkw), _full(wkb),
                  _full(gw), _full(As), _full(Ad)],
        out_specs=[_row(64), _row(32), _row(32), _row(32), _row(32),
                   _row(16), _row(16)],
        out_shape=[f((n, 64), jnp.float32), f((n, 32), jnp.float32),
                   f((n, 32), jnp.float32), f((n, 32), jnp.float32),
                   f((n, 32), jnp.float32), f((n, 16), jnp.float32),
                   f((n, 16), jnp.float32)],
    )(*args)


def _mid_body(h_ref, a0_ref, a1_ref, d0_ref, d1_ref, slw, slb, srw,
              sage_ref, dinv_ref):
    dd = d0_ref[...] + d1_ref[...]
    deg = jnp.clip(dd[:, 4:5], 1.0, None)
    dinv_ref[...] = 1.0 / jnp.clip(dd, 1e-16, None)
    agg = jnp.concatenate([a0_ref[...], a1_ref[...]], axis=-1) / deg
    sage_ref[...] = jnp.maximum(
        agg @ slw[...] + slb[...] + h_ref[...] @ srw[...], 0.0)


def _tc_mid(h, a0, a1, d0, d1, slw, slb, srw):
    n = h.shape[0]
    f = jax.ShapeDtypeStruct
    return pl.pallas_call(
        _mid_body,
        grid=(n // BN,),
        in_specs=[_row(64), _row(32), _row(32), _row(16), _row(16),
                  _full(slw), _full(slb), _full(srw)],
        out_specs=[_row(64), _row(16)],
        out_shape=[f((n, 64), jnp.float32), f((n, 16), jnp.float32)],
    )(h, a0, a1, d0, d1, slw, slb, srw)


def _post_ln(sage, att0, att1, h, gatb):
    att = jnp.concatenate([att0, att1], axis=-1) + gatb
    hh = sage + att + h
    mu = jnp.mean(hh, -1, keepdims=True)
    var = jnp.mean((hh - mu) ** 2, -1, keepdims=True)
    return (hh - mu) * lax.rsqrt(var + 1e-5)


def _post0_body(sage_ref, att0_ref, att1_ref, h_ref, gatb, gw, As, Ad,
                h_ref_o, hlo_ref, hhi_ref, glo_ref, ghi_ref, as_ref, ad_ref):
    hn = _post_ln(sage_ref[...], att0_ref[...], att1_ref[...], h_ref[...],
                  gatb[...])
    h_ref_o[...] = hn
    hlo_ref[...] = hn[:, :32]
    hhi_ref[...] = hn[:, 32:]
    g = hn @ gw[...]
    glo_ref[...] = g[:, :32]
    ghi_ref[...] = g[:, 32:]
    zs = jnp.zeros((hn.shape[0], 12), jnp.float32)
    as_ref[...] = jnp.concatenate([g @ As[...], zs], axis=1)
    ad_ref[...] = jnp.concatenate([g @ Ad[...], zs], axis=1)


def _tc_post0(sage, att0, att1, h, gatb, gw, As, Ad):
    n = h.shape[0]
    f = jax.ShapeDtypeStruct
    return pl.pallas_call(
        _post0_body,
        grid=(n // BN,),
        in_specs=[_row(64), _row(32), _row(32), _row(64),
                  _full(gatb), _full(gw), _full(As), _full(Ad)],
        out_specs=[_row(64), _row(32), _row(32), _row(32), _row(32),
                   _row(16), _row(16)],
        out_shape=[f((n, 64), jnp.float32), f((n, 32), jnp.float32),
                   f((n, 32), jnp.float32), f((n, 32), jnp.float32),
                   f((n, 32), jnp.float32), f((n, 16), jnp.float32),
                   f((n, 16), jnp.float32)],
    )(sage, att0, att1, h, gatb, gw, As, Ad)


def _post1_body(sage_ref, att0_ref, att1_ref, h_ref, gatb, we_ref):
    we_ref[...] = _post_ln(sage_ref[...], att0_ref[...], att1_ref[...],
                           h_ref[...], gatb[...])


def _tc_post1(sage, att0, att1, h, gatb):
    n = h.shape[0]
    return pl.pallas_call(
        _post1_body,
        grid=(n // BN,),
        in_specs=[_row(64), _row(32), _row(32), _row(64), _full(gatb)],
        out_specs=_row(64),
        out_shape=jax.ShapeDtypeStruct((n, 64), jnp.float32),
    )(sage, att0, att1, h, gatb)


def _heads_body(we_ref, r1w, r1b, r2w, r2b, r3w, r3b, c1w, c1b, c2w, c2b,
                e1w, e1b, e2w, e2b, out_ref):
    we = we_ref[...]
    r = jnp.maximum(we @ r1w[...] + r1b[...], 0.0)
    r = jnp.maximum(r @ r2w[...] + r2b[...], 0.0)
    res = jax.nn.sigmoid(r @ r3w[...] + r3b[...])
    c = jnp.maximum(we @ c1w[...] + c1b[...], 0.0)
    conf = jax.nn.sigmoid(c @ c2w[...] + c2b[...])
    ev = jnp.maximum(we @ e1w[...] + e1b[...], 0.0)
    z = ev @ e2w[...] + e2b[...]
    z = z - jnp.max(z, axis=-1, keepdims=True)
    pz = jnp.exp(z)
    pz = pz / jnp.sum(pz, axis=-1, keepdims=True)
    out_ref[:, 0:1] = res
    out_ref[:, 1:2] = conf
    out_ref[:, 2:7] = pz
    out_ref[:, 7:8] = res


def _heads(we, p):
    n = we.shape[0]
    args = [
        p['r1_W'], p['r1_b'].reshape(1, -1),
        p['r2_W'], p['r2_b'].reshape(1, -1),
        p['r3_W'], p['r3_b'].reshape(1, -1),
        p['c1_W'], p['c1_b'].reshape(1, -1),
        p['c2_W'], p['c2_b'].reshape(1, -1),
        p['e1_W'], p['e1_b'].reshape(1, -1),
        p['e2_W'], p['e2_b'].reshape(1, -1),
    ]
    in_specs = [_row(H)] + [_full(a) for a in args]
    out = pl.pallas_call(
        _heads_body,
        grid=(n // BN,),
        in_specs=in_specs,
        out_specs=_row(8),
        out_shape=jax.ShapeDtypeStruct((n, 8), jnp.float32),
    )(we, *args)
    return out[:, 0], out[:, 1], out[:, 2:7]


def kernel(x, edge_index, edge_attr, node_type, edge_type, params):
    p = params
    n = x.shape[0]
    src = edge_index[0].astype(jnp.int32)
    dst = edge_index[1].astype(jnp.int32)

    def block_diag_a(a):
        return (jnp.eye(HEADS, dtype=jnp.float32)[:, None, :]
                * a[:, :, None]).reshape(H, HEADS)

    As = [block_diag_a(p[f'gat_as_{i}']) for i in range(L)]
    Ad = [block_diag_a(p[f'gat_ad_{i}']) for i in range(L)]

    h, h_lo, h_hi, g_lo, g_hi, as16, ad16 = _tc_prep(
        x, node_type.reshape(n, 1), p['wk_W'], p['wk_b'].reshape(1, -1),
        p['gat_W_0'], As[0], Ad[0])

    we = None
    for i in range(L):
        aggh = _sc_agg(h_lo, h_hi, src, dst)
        ex16, ddp = _sc_attstats(as16, ad16, src, dst)
        sage, dinv16 = _tc_mid(h, aggh[0], aggh[1], ddp[0], ddp[1],
                               p[f'sage_l_W_{i}'],
                               p[f'sage_l_b_{i}'].reshape(1, -1),
                               p[f'sage_r_W_{i}'])
        atth = _sc_att(g_lo, g_hi, dinv16, ex16, src, dst)
        gatb = p[f'gat_b_{i}'].reshape(1, -1)
        if i == 0:
            h, h_lo, h_hi, g_lo, g_hi, as16, ad16 = _tc_post0(
                sage, atth[0], atth[1], h, gatb, p['gat_W_1'], As[1], Ad[1])
        else:
            we = _tc_post1(sage, atth[0], atth[1], h, gatb)

    resilience, confidence, explanation = _heads(we, p)
    return resilience, confidence, explanation, we


# speedup vs baseline: 49.8775x; 1.4249x over previous
import functools

import jax
import jax.numpy as jnp
from jax import lax
from jax.experimental import pallas as pl
from jax.experimental.pallas import tpu as pltpu
from jax.experimental.pallas import tpu_sc as plsc

H = 64
HEADS = 4
HD = 16
L = 2
NCORES = 2
NTILES = 16
BN = 2000
KW = 200
ZR = 112


def _mesh():
    return plsc.VectorSubcoreMesh(core_axis_name="c", subcore_axis_name="s")


def _npad(n):
    return -(-n // (NTILES * ZR)) * (NTILES * ZR)


def _sc_agg(h_lo, h_hi, src, dst):
    n = h_lo.shape[0]
    e = src.shape[0]
    pte = e // NTILES
    K = KW
    nwin = pte // K
    npairs = nwin // 2
    tail = nwin % 2
    npad = _npad(n)
    ptn = npad // NTILES
    nz = ptn // ZR

    @functools.partial(
        pl.kernel, mesh=_mesh(),
        compiler_params=pltpu.CompilerParams(use_tc_tiling_on_sc=False),
        out_type=jax.ShapeDtypeStruct((NCORES, npad, 32), jnp.float32),
        scratch_types=[
            pltpu.VMEM((K,), jnp.int32), pltpu.VMEM((K,), jnp.int32),
            pltpu.VMEM((K,), jnp.int32), pltpu.VMEM((K,), jnp.int32),
            pltpu.VMEM((K, 32), jnp.float32), pltpu.VMEM((K, 32), jnp.float32),
            pltpu.VMEM_SHARED((npad, 32), jnp.float32),
            pltpu.SemaphoreType.DMA, pltpu.SemaphoreType.DMA,
        ],
    )
    def k(hlo_h, hhi_h, src_h, dst_h, out_h,
          siA, diA, siB, diB, rowsA, rowsB, acc, semA, semB):
        core = lax.axis_index("c")
        sid = lax.axis_index("s")

        def zb(i, carry):
            rowsA[i, pl.ds(0, 16)] = jnp.zeros((16,), jnp.float32)
            rowsA[i, pl.ds(16, 16)] = jnp.zeros((16,), jnp.float32)
            return carry

        lax.fori_loop(0, ZR, zb, 0)
        r0 = sid * ptn
        for z in range(nz):
            pltpu.sync_copy(rowsA.at[pl.ds(0, ZR)],
                            acc.at[pl.ds(r0 + z * ZR, ZR)])
        plsc.subcore_barrier()
        base = sid * pte

        def issue(b, si, di, rows, sem):
            pltpu.sync_copy(src_h.at[pl.ds(b, K)], si)
            pltpu.sync_copy(dst_h.at[pl.ds(b, K)], di)

            @pl.when(core == 0)
            def _():
                pltpu.async_copy(hlo_h.at[si], rows, sem)

            @pl.when(core == 1)
            def _():
                pltpu.async_copy(hhi_h.at[si], rows, sem)

        def wait(rows, sem):
            pltpu.make_async_copy(hlo_h.at[pl.ds(0, K)], rows, sem).wait()

        issue(base, siA, diA, rowsA, semA)

        def pair(i, carry):
            w0 = base + (2 * i) * K
            wait(rowsA, semA)
            issue(w0 + K, siB, diB, rowsB, semB)
            pltpu.sync_copy(rowsA, acc.at[diA], add=True)
            wait(rowsB, semB)

            @pl.when(2 * i + 2 < nwin)
            def _():
                issue(w0 + 2 * K, siA, diA, rowsA, semA)

            pltpu.sync_copy(rowsB, acc.at[diB], add=True)
            return carry

        lax.fori_loop(0, npairs, pair, 0)
        if tail:
            wait(rowsA, semA)
            pltpu.sync_copy(rowsA, acc.at[diA], add=True)
        plsc.subcore_barrier()
        for z in range(nz):
            rr = r0 + z * ZR
            pltpu.sync_copy(acc.at[pl.ds(rr, ZR)], rowsA.at[pl.ds(0, ZR)])
            pltpu.sync_copy(rowsA.at[pl.ds(0, ZR)],
                            out_h.at[core, pl.ds(rr, ZR)])

    return k(h_lo, h_hi, src, dst)


def _sc_attstats(as16, ad16, src, dst):
    n = as16.shape[0]
    e = src.shape[0]
    he = e // NCORES
    pte = he // NTILES
    K = KW
    nwin = pte // K
    npairs = nwin // 2
    tail = nwin % 2
    npad = _npad(n)
    ptn = npad // NTILES
    nz = ptn // ZR

    @functools.partial(
        pl.kernel, mesh=_mesh(),
        compiler_params=pltpu.CompilerParams(use_tc_tiling_on_sc=False),
        out_type=[
            jax.ShapeDtypeStruct((e, 16), jnp.float32),
            jax.ShapeDtypeStruct((NCORES, npad, 16), jnp.float32),
        ],
        scratch_types=[
            pltpu.VMEM((K,), jnp.int32), pltpu.VMEM((K,), jnp.int32),
            pltpu.VMEM((K,), jnp.int32), pltpu.VMEM((K,), jnp.int32),
            pltpu.VMEM((K, 16), jnp.float32), pltpu.VMEM((K, 16), jnp.float32),
            pltpu.VMEM((K, 16), jnp.float32), pltpu.VMEM((K, 16), jnp.float32),
            pltpu.VMEM((K, 16), jnp.float32), pltpu.VMEM((K, 16), jnp.float32),
            pltpu.VMEM_SHARED((npad, 16), jnp.float32),
            pltpu.SemaphoreType.DMA, pltpu.SemaphoreType.DMA,
        ],
    )
    def k(as_h, ad_h, src_h, dst_h, ex_h, ddp_h,
          siA, diA, siB, diB, asbA, adbA, exbA, asbB, adbB, exbB,
          acc, semA, semB):
        core = lax.axis_index("c")
        sid = lax.axis_index("s")

        def zb(i, carry):
            asbA[i, :] = jnp.zeros((16,), jnp.float32)
            return carry

        lax.fori_loop(0, ZR, zb, 0)
        r0 = sid * ptn
        for z in range(nz):
            pltpu.sync_copy(asbA.at[pl.ds(0, ZR)],
                            acc.at[pl.ds(r0 + z * ZR, ZR)])
        plsc.subcore_barrier()
        base = core * he + sid * pte

        def issue(b, si, di, asb, adb, sem):
            pltpu.sync_copy(src_h.at[pl.ds(b, K)], si)
            pltpu.sync_copy(dst_h.at[pl.ds(b, K)], di)
            pltpu.async_copy(as_h.at[si], asb, sem)
            pltpu.async_copy(ad_h.at[di], adb, sem)

        def wait(asb, adb, sem):
            pltpu.make_async_copy(as_h.at[pl.ds(0, K)], asb, sem).wait()
            pltpu.make_async_copy(ad_h.at[pl.ds(0, K)], adb, sem).wait()

        def compute(asb, adb, exb):
            def body(j, carry):
                xv = asb[j, :] + adb[j, :]
                exb[j, :] = jnp.exp(jnp.maximum(xv, 0.2 * xv))
                return carry

            lax.fori_loop(0, K, body, 0)

        def scatter(b, di, exb):
            pltpu.sync_copy(exb, acc.at[di], add=True)
            pltpu.sync_copy(exb, ex_h.at[pl.ds(b, K)])

        issue(base, siA, diA, asbA, adbA, semA)

        def pair(i, carry):
            w0 = base + (2 * i) * K
            wait(asbA, adbA, semA)
            issue(w0 + K, siB, diB, asbB, adbB, semB)
            compute(asbA, adbA, exbA)
            scatter(w0, diA, exbA)
            wait(asbB, adbB, semB)

            @pl.when(2 * i + 2 < nwin)
            def _():
                issue(w0 + 2 * K, siA, diA, asbA, adbA, semA)

            compute(asbB, adbB, exbB)
            scatter(w0 + K, diB, exbB)
            return carry

        lax.fori_loop(0, npairs, pair, 0)
        if tail:
            wait(asbA, adbA, semA)
            compute(asbA, adbA, exbA)
            scatter(base + (nwin - 1) * K, diA, exbA)
        plsc.subcore_barrier()
        for z in range(nz):
            rr = r0 + z * ZR
            pltpu.sync_copy(acc.at[pl.ds(rr, ZR)], asbA.at[pl.ds(0, ZR)])
            pltpu.sync_copy(asbA.at[pl.ds(0, ZR)],
                            ddp_h.at[core, pl.ds(rr, ZR)])

    return k(as16, ad16, src, dst)


_GD = lax.GatherDimensionNumbers(offset_dims=(), collapsed_slice_dims=(0,),
                                 start_index_map=(0,))


def _lane_gather(v, idx):
    return lax.gather(v, idx[:, None], _GD, (1,),
                      mode=lax.GatherScatterMode.PROMISE_IN_BOUNDS)


def _sc_att(g_lo, g_hi, dinv16, ex16, src, dst):
    n = g_lo.shape[0]
    e = src.shape[0]
    pte = e // NTILES
    K = KW
    nwin = pte // K
    npairs = nwin // 2
    tail = nwin % 2
    npad = _npad(n)
    ptn = npad // NTILES
    nz = ptn // ZR

    @functools.partial(
        pl.kernel, mesh=_mesh(),
        compiler_params=pltpu.CompilerParams(use_tc_tiling_on_sc=False),
        out_type=jax.ShapeDtypeStruct((NCORES, npad, 32), jnp.float32),
        scratch_types=[
            pltpu.VMEM((K,), jnp.int32), pltpu.VMEM((K,), jnp.int32),
            pltpu.VMEM((K,), jnp.int32), pltpu.VMEM((K,), jnp.int32),
            pltpu.VMEM((K, 32), jnp.float32), pltpu.VMEM((K, 32), jnp.float32),
            pltpu.VMEM((K, 16), jnp.float32), pltpu.VMEM((K, 16), jnp.float32),
            pltpu.VMEM((K, 16), jnp.float32), pltpu.VMEM((K, 16), jnp.float32),
            pltpu.VMEM_SHARED((npad, 32), jnp.float32),
            pltpu.SemaphoreType.DMA, pltpu.SemaphoreType.DMA,
        ],
    )
    def k(glo_h, ghi_h, dinv_h, ex_h, src_h, dst_h, out_h,
          siA, diA, siB, diB, growA, growB, dvbA, dvbB, exbA, exbB,
          acc, semA, semB):
        core = lax.axis_index("c")
        sid = lax.axis_index("s")
        i0 = jnp.full((16,), core * 2, jnp.int32)
        i1 = jnp.full((16,), core * 2 + 1, jnp.int32)

        def zb(i, carry):
            growA[i, pl.ds(0, 16)] = jnp.zeros((16,), jnp.float32)
            growA[i, pl.ds(16, 16)] = jnp.zeros((16,), jnp.float32)
            return carry

        lax.fori_loop(0, ZR, zb, 0)
        r0 = sid * ptn
        for z in range(nz):
            pltpu.sync_copy(growA.at[pl.ds(0, ZR)],
                            acc.at[pl.ds(r0 + z * ZR, ZR)])
        plsc.subcore_barrier()
        base = sid * pte

        def issue(b, si, di, grow, dvb, exb, sem):
            pltpu.sync_copy(src_h.at[pl.ds(b, K)], si)
            pltpu.sync_copy(dst_h.at[pl.ds(b, K)], di)

            @pl.when(core == 0)
            def _():
                pltpu.async_copy(glo_h.at[si], grow, sem)

            @pl.when(core == 1)
            def _():
                pltpu.async_copy(ghi_h.at[si], grow, sem)

            pltpu.async_copy(dinv_h.at[di], dvb, sem)
            pltpu.async_copy(ex_h.at[pl.ds(b, K)], exb, sem)

        def wait(grow, dvb, exb, sem):
            pltpu.make_async_copy(glo_h.at[pl.ds(0, K)], grow, sem).wait()
            pltpu.make_async_copy(dinv_h.at[pl.ds(0, K)], dvb, sem).wait()
            pltpu.make_async_copy(ex_h.at[pl.ds(0, K)], exb, sem).wait()

        def compute(grow, dvb, exb):
            def body(j, carry):
                wv = exb[j, :] * dvb[j, :]
                w0 = _lane_gather(wv, i0)
                w1 = _lane_gather(wv, i1)
                grow[j, pl.ds(0, 16)] = grow[j, pl.ds(0, 16)] * w0
                grow[j, pl.ds(16, 16)] = grow[j, pl.ds(16, 16)] * w1
                return carry

            lax.fori_loop(0, K, body, 0)

        issue(base, siA, diA, growA, dvbA, exbA, semA)

        def pair(i, carry):
            w0 = base + (2 * i) * K
            wait(growA, dvbA, exbA, semA)
            issue(w0 + K, siB, diB, growB, dvbB, exbB, semB)
            compute(growA, dvbA, exbA)
            pltpu.sync_copy(growA, acc.at[diA], add=True)
            wait(growB, dvbB, exbB, semB)

            @pl.when(2 * i + 2 < nwin)
            def _():
                issue(w0 + 2 * K, siA, diA, growA, dvbA, exbA, semA)

            compute(growB, dvbB, exbB)
            pltpu.sync_copy(growB, acc.at[diB], add=True)
            return carry

        lax.fori_loop(0, npairs, pair, 0)
        if tail:
            wait(growA, dvbA, exbA, semA)
            compute(growA, dvbA, exbA)
            pltpu.sync_copy(growA, acc.at[diA], add=True)
        plsc.subcore_barrier()
        for z in range(nz):
            rr = r0 + z * ZR
            pltpu.sync_copy(acc.at[pl.ds(rr, ZR)], growA.at[pl.ds(0, ZR)])
            pltpu.sync_copy(growA.at[pl.ds(0, ZR)],
                            out_h.at[core, pl.ds(rr, ZR)])

    return k(g_lo, g_hi, dinv16, ex16, src, dst)


def _full(a):
    return pl.BlockSpec(a.shape, lambda i: tuple(0 for _ in a.shape))


def _row(width):
    return pl.BlockSpec((BN, width), lambda i: (i, 0))


def _prep_body(x_ref, nt_ref, wkw, wkb, gw, As, Ad,
               h_ref, hlo_ref, hhi_ref, glo_ref, ghi_ref, as_ref, ad_ref):
    m = (nt_ref[...] == 0).astype(jnp.float32)
    h = (x_ref[...] @ wkw[...] + wkb[...]) * m
    h_ref[...] = h
    hlo_ref[...] = h[:, :32]
    hhi_ref[...] = h[:, 32:]
    g = h @ gw[...]
    glo_ref[...] = g[:, :32]
    ghi_ref[...] = g[:, 32:]
    zs = jnp.zeros((h.shape[0], 12), jnp.float32)
    as_ref[...] = jnp.concatenate([g @ As[...], zs], axis=1)
    ad_ref[...] = jnp.concatenate([g @ Ad[...], zs], axis=1)


def _tc_prep(x, nt2, wkw, wkb, gw, As, Ad):
    n = x.shape[0]
    f = jax.ShapeDtypeStruct
    args = (x, nt2, wkw, wkb, gw, As, Ad)
    return pl.pallas_call(
        _prep_body,
        grid=(n // BN,),
        in_specs=[_row(x.shape[1]), _row(1), _full(wkw), _full(wkb),
                  _full(gw), _full(As), _full(Ad)],
        out_specs=[_row(64), _row(32), _row(32), _row(32), _row(32),
                   _row(16), _row(16)],
        out_shape=[f((n, 64), jnp.float32), f((n, 32), jnp.float32),
                   f((n, 32), jnp.float32), f((n, 32), jnp.float32),
                   f((n, 32), jnp.float32), f((n, 16), jnp.float32),
                   f((n, 16), jnp.float32)],
    )(*args)


def _mid_body(h_ref, a0_ref, a1_ref, d0_ref, d1_ref, slw, slb, srw,
              sage_ref, dinv_ref):
    dd = d0_ref[...] + d1_ref[...]
    deg = jnp.clip(dd[:, 4:5], 1.0, None)
    dinv_ref[...] = 1.0 / jnp.clip(dd, 1e-16, None)
    agg = jnp.concatenate([a0_ref[...], a1_ref[...]], axis=-1) / deg
    sage_ref[...] = jnp.maximum(
        agg @ slw[...] + slb[...] + h_ref[...] @ srw[...], 0.0)


def _tc_mid(h, a0, a1, d0, d1, slw, slb, srw):
    n = h.shape[0]
    f = jax.ShapeDtypeStruct
    return pl.pallas_call(
        _mid_body,
        grid=(n // BN,),
        in_specs=[_row(64), _row(32), _row(32), _row(16), _row(16),
                  _full(slw), _full(slb), _full(srw)],
        out_specs=[_row(64), _row(16)],
        out_shape=[f((n, 64), jnp.float32), f((n, 16), jnp.float32)],
    )(h, a0, a1, d0, d1, slw, slb, srw)


def _post_ln(sage, att0, att1, h, gatb):
    att = jnp.concatenate([att0, att1], axis=-1) + gatb
    hh = sage + att + h
    mu = jnp.mean(hh, -1, keepdims=True)
    var = jnp.mean((hh - mu) ** 2, -1, keepdims=True)
    return (hh - mu) * lax.rsqrt(var + 1e-5)


def _post0_body(sage_ref, att0_ref, att1_ref, h_ref, gatb, gw, As, Ad,
                h_ref_o, hlo_ref, hhi_ref, glo_ref, ghi_ref, as_ref, ad_ref):
    hn = _post_ln(sage_ref[...], att0_ref[...], att1_ref[...], h_ref[...],
                  gatb[...])
    h_ref_o[...] = hn
    hlo_ref[...] = hn[:, :32]
    hhi_ref[...] = hn[:, 32:]
    g = hn @ gw[...]
    glo_ref[...] = g[:, :32]
    ghi_ref[...] = g[:, 32:]
    zs = jnp.zeros((hn.shape[0], 12), jnp.float32)
    as_ref[...] = jnp.concatenate([g @ As[...], zs], axis=1)
    ad_ref[...] = jnp.concatenate([g @ Ad[...], zs], axis=1)


def _tc_post0(sage, att0, att1, h, gatb, gw, As, Ad):
    n = h.shape[0]
    f = jax.ShapeDtypeStruct
    return pl.pallas_call(
        _post0_body,
        grid=(n // BN,),
        in_specs=[_row(64), _row(32), _row(32), _row(64),
                  _full(gatb), _full(gw), _full(As), _full(Ad)],
        out_specs=[_row(64), _row(32), _row(32), _row(32), _row(32),
                   _row(16), _row(16)],
        out_shape=[f((n, 64), jnp.float32), f((n, 32), jnp.float32),
                   f((n, 32), jnp.float32), f((n, 32), jnp.float32),
                   f((n, 32), jnp.float32), f((n, 16), jnp.float32),
                   f((n, 16), jnp.float32)],
    )(sage, att0, att1, h, gatb, gw, As, Ad)


def _post1_body(sage_ref, att0_ref, att1_ref, h_ref, gatb, we_ref):
    we_ref[...] = _post_ln(sage_ref[...], att0_ref[...], att1_ref[...],
                           h_ref[...], gatb[...])


def _tc_post1(sage, att0, att1, h, gatb):
    n = h.shape[0]
    return pl.pallas_call(
        _post1_body,
        grid=(n // BN,),
        in_specs=[_row(64), _row(32), _row(32), _row(64), _full(gatb)],
        out_specs=_row(64),
        out_shape=jax.ShapeDtypeStruct((n, 64), jnp.float32),
    )(sage, att0, att1, h, gatb)


def _heads_body(we_ref, r1w, r1b, r2w, r2b, r3w, r3b, c1w, c1b, c2w, c2b,
                e1w, e1b, e2w, e2b, out_ref):
    we = we_ref[...]
    r = jnp.maximum(we @ r1w[...] + r1b[...], 0.0)
    r = jnp.maximum(r @ r2w[...] + r2b[...], 0.0)
    res = jax.nn.sigmoid(r @ r3w[...] + r3b[...])
    c = jnp.maximum(we @ c1w[...] + c1b[...], 0.0)
    conf = jax.nn.sigmoid(c @ c2w[...] + c2b[...])
    ev = jnp.maximum(we @ e1w[...] + e1b[...], 0.0)
    z = ev @ e2w[...] + e2b[...]
    z = z - jnp.max(z, axis=-1, keepdims=True)
    pz = jnp.exp(z)
    pz = pz / jnp.sum(pz, axis=-1, keepdims=True)
    out_ref[:, 0:1] = res
    out_ref[:, 1:2] = conf
    out_ref[:, 2:7] = pz
    out_ref[:, 7:8] = res


def _heads(we, p):
    n = we.shape[0]
    args = [
        p['r1_W'], p['r1_b'].reshape(1, -1),
        p['r2_W'], p['r2_b'].reshape(1, -1),
        p['r3_W'], p['r3_b'].reshape(1, -1),
        p['c1_W'], p['c1_b'].reshape(1, -1),
        p['c2_W'], p['c2_b'].reshape(1, -1),
        p['e1_W'], p['e1_b'].reshape(1, -1),
        p['e2_W'], p['e2_b'].reshape(1, -1),
    ]
    in_specs = [_row(H)] + [_full(a) for a in args]
    out = pl.pallas_call(
        _heads_body,
        grid=(n // BN,),
        in_specs=in_specs,
        out_specs=_row(8),
        out_shape=jax.ShapeDtypeStruct((n, 8), jnp.float32),
    )(we, *args)
    return out[:, 0], out[:, 1], out[:, 2:7]


def kernel(x, edge_index, edge_attr, node_type, edge_type, params):
    p = params
    n = x.shape[0]
    src = edge_index[0].astype(jnp.int32)
    dst = edge_index[1].astype(jnp.int32)

    def block_diag_a(a):
        return (jnp.eye(HEADS, dtype=jnp.float32)[:, None, :]
                * a[:, :, None]).reshape(H, HEADS)

    As = [block_diag_a(p[f'gat_as_{i}']) for i in range(L)]
    Ad = [block_diag_a(p[f'gat_ad_{i}']) for i in range(L)]

    h, h_lo, h_hi, g_lo, g_hi, as16, ad16 = _tc_prep(
        x, node_type.reshape(n, 1), p['wk_W'], p['wk_b'].reshape(1, -1),
        p['gat_W_0'], As[0], Ad[0])

    we = None
    for i in range(L):
        aggh = _sc_agg(h_lo, h_hi, src, dst)
        ex16, ddp = _sc_attstats(as16, ad16, src, dst)
        sage, dinv16 = _tc_mid(h, aggh[0], aggh[1], ddp[0], ddp[1],
                               p[f'sage_l_W_{i}'],
                               p[f'sage_l_b_{i}'].reshape(1, -1),
                               p[f'sage_r_W_{i}'])
        atth = _sc_att(g_lo, g_hi, dinv16, ex16, src, dst)
        gatb = p[f'gat_b_{i}'].reshape(1, -1)
        if i == 0:
            h, h_lo, h_hi, g_lo, g_hi, as16, ad16 = _tc_post0(
                sage, atth[0], atth[1], h, gatb, p['gat_W_1'], As[1], Ad[1])
        else:
            we = _tc_post1(sage, atth[0], atth[1], h, gatb)

    resilience, confidence, explanation = _heads(we, p)
    return resilience, confidence, explanation, we


# speedup vs baseline: 64.8783x; 1.3008x over previous
import functools

import jax
import jax.numpy as jnp
from jax import lax
from jax.experimental import pallas as pl
from jax.experimental.pallas import tpu as pltpu
from jax.experimental.pallas import tpu_sc as plsc

H = 64
HEADS = 4
HD = 16
L = 2
NCORES = 2
NTILES = 16
BN = 2000
KW = 200
ZR = 112


def _mesh():
    return plsc.VectorSubcoreMesh(core_axis_name="c", subcore_axis_name="s")


def _npad(n):
    return -(-n // (NTILES * ZR)) * (NTILES * ZR)


def _sc_agg(h_lo, h_hi, src2, dst2):
    n = h_lo.shape[0]
    K = KW
    nrows = src2.shape[0]
    ptr = nrows // NTILES
    IT = 10
    nit = ptr // IT
    tail = ptr % IT
    npad = _npad(n)
    ptn = npad // NTILES
    nz = ptn // ZR

    @functools.partial(
        pl.kernel, mesh=_mesh(),
        compiler_params=pltpu.CompilerParams(use_tc_tiling_on_sc=False),
        out_type=jax.ShapeDtypeStruct((NCORES, npad, 32), jnp.float32),
        scratch_types=[
            pltpu.VMEM((IT, KW), jnp.int32), pltpu.VMEM((IT, KW), jnp.int32),
            pltpu.VMEM((KW, 32), jnp.float32), pltpu.VMEM((KW, 32), jnp.float32),
            pltpu.VMEM_SHARED((npad, 32), jnp.float32),
            pltpu.SemaphoreType.DMA, pltpu.SemaphoreType.DMA,
        ],
    )
    def k(hlo_h, hhi_h, src_h, dst_h, zz_h, out_h,
          sib, dib, rowsA, rowsB, acc, semA, semB):
        core = lax.axis_index("c")
        sid = lax.axis_index("s")
        r0 = sid * ptn
        pltpu.sync_copy(zz_h.at[pl.ds(r0, ptn)], acc.at[pl.ds(r0, ptn)])
        plsc.subcore_barrier()
        wrow0 = sid * ptr

        def issue(j, rows, sem):
            @pl.when(core == 0)
            def _():
                pltpu.async_copy(hlo_h.at[sib.at[j]], rows, sem)

            @pl.when(core == 1)
            def _():
                pltpu.async_copy(hhi_h.at[sib.at[j]], rows, sem)

        def wait(rows, sem):
            pltpu.make_async_copy(hlo_h.at[pl.ds(0, K)], rows, sem).wait()

        bufs = ((rowsA, semA), (rowsB, semB))

        def run_group(nw):
            issue(0, *bufs[0])
            for j in range(nw):
                rows, sem = bufs[j % 2]
                wait(rows, sem)
                if j + 1 < nw:
                    issue(j + 1, *bufs[(j + 1) % 2])
                pltpu.sync_copy(rows, acc.at[dib.at[j]], add=True)

        def it(i, carry):
            wr = wrow0 + IT * i
            pltpu.sync_copy(src_h.at[pl.ds(wr, IT)], sib)
            pltpu.sync_copy(dst_h.at[pl.ds(wr, IT)], dib)
            run_group(IT)
            return carry

        lax.fori_loop(0, nit, it, 0)
        if tail:
            wr = wrow0 + IT * nit
            pltpu.sync_copy(src_h.at[pl.ds(wr, tail)], sib.at[pl.ds(0, tail)])
            pltpu.sync_copy(dst_h.at[pl.ds(wr, tail)], dib.at[pl.ds(0, tail)])
            run_group(tail)
        plsc.subcore_barrier()
        pltpu.sync_copy(acc.at[pl.ds(r0, ptn)], out_h.at[core, pl.ds(r0, ptn)])

    return k(h_lo, h_hi, src2, dst2, jnp.zeros((npad, 32), jnp.float32))


def _sc_attstats(as16, ad16, src2, dst2):
    n = as16.shape[0]
    K = KW
    nrows = src2.shape[0]
    e = nrows * K
    hr = nrows // NCORES
    ptr = hr // NTILES
    IT = 10
    nit = ptr // IT
    tail = ptr % IT
    npad = _npad(n)
    ptn = npad // NTILES
    nz = ptn // ZR

    @functools.partial(
        pl.kernel, mesh=_mesh(),
        compiler_params=pltpu.CompilerParams(use_tc_tiling_on_sc=False),
        out_type=[
            jax.ShapeDtypeStruct((e, 16), jnp.float32),
            jax.ShapeDtypeStruct((NCORES, npad, 16), jnp.float32),
        ],
        scratch_types=[
            pltpu.VMEM((IT, KW), jnp.int32), pltpu.VMEM((IT, KW), jnp.int32),
            pltpu.VMEM((KW, 16), jnp.float32), pltpu.VMEM((KW, 16), jnp.float32),
            pltpu.VMEM((KW, 16), jnp.float32), pltpu.VMEM((KW, 16), jnp.float32),
            pltpu.VMEM((KW, 16), jnp.float32), pltpu.VMEM((KW, 16), jnp.float32),
            pltpu.VMEM_SHARED((npad, 16), jnp.float32),
            pltpu.SemaphoreType.DMA, pltpu.SemaphoreType.DMA,
        ],
    )
    def k(as_h, ad_h, src_h, dst_h, zz_h, ex_h, ddp_h,
          sib, dib, asbA, adbA, exbA, asbB, adbB, exbB, acc, semA, semB):
        core = lax.axis_index("c")
        sid = lax.axis_index("s")
        r0 = sid * ptn
        pltpu.sync_copy(zz_h.at[pl.ds(r0, ptn)], acc.at[pl.ds(r0, ptn)])
        plsc.subcore_barrier()
        wrow0 = core * hr + sid * ptr

        def issue(j, asb, adb, sem):
            pltpu.async_copy(as_h.at[sib.at[j]], asb, sem)
            pltpu.async_copy(ad_h.at[dib.at[j]], adb, sem)

        def wait(asb, adb, sem):
            pltpu.make_async_copy(as_h.at[pl.ds(0, K)], asb, sem).wait()
            pltpu.make_async_copy(ad_h.at[pl.ds(0, K)], adb, sem).wait()

        def compute(asb, adb, exb):
            def body(j, carry):
                xv = asb[j, :] + adb[j, :]
                exb[j, :] = jnp.exp(jnp.maximum(xv, 0.2 * xv))
                return carry

            lax.fori_loop(0, K, body, 0)

        bufs = ((asbA, adbA, exbA, semA), (asbB, adbB, exbB, semB))

        def run_group(nw, wr):
            issue(0, bufs[0][0], bufs[0][1], bufs[0][3])
            for j in range(nw):
                asb, adb, exb, sem = bufs[j % 2]
                wait(asb, adb, sem)
                if j + 1 < nw:
                    nxt = bufs[(j + 1) % 2]
                    issue(j + 1, nxt[0], nxt[1], nxt[3])
                compute(asb, adb, exb)
                pltpu.sync_copy(exb, acc.at[dib.at[j]], add=True)
                pltpu.sync_copy(exb, ex_h.at[pl.ds((wr + j) * K, K)])

        def it(i, carry):
            wr = wrow0 + IT * i
            pltpu.sync_copy(src_h.at[pl.ds(wr, IT)], sib)
            pltpu.sync_copy(dst_h.at[pl.ds(wr, IT)], dib)
            run_group(IT, wr)
            return carry

        lax.fori_loop(0, nit, it, 0)
        if tail:
            wr = wrow0 + IT * nit
            pltpu.sync_copy(src_h.at[pl.ds(wr, tail)], sib.at[pl.ds(0, tail)])
            pltpu.sync_copy(dst_h.at[pl.ds(wr, tail)], dib.at[pl.ds(0, tail)])
            run_group(tail, wr)
        plsc.subcore_barrier()
        pltpu.sync_copy(acc.at[pl.ds(r0, ptn)], ddp_h.at[core, pl.ds(r0, ptn)])

    return k(as16, ad16, src2, dst2, jnp.zeros((npad, 16), jnp.float32))


_GD = lax.GatherDimensionNumbers(offset_dims=(), collapsed_slice_dims=(0,),
                                 start_index_map=(0,))


def _lane_gather(v, idx):
    return lax.gather(v, idx[:, None], _GD, (1,),
                      mode=lax.GatherScatterMode.PROMISE_IN_BOUNDS)


def _sc_att(g_lo, g_hi, dinv16, ex16, src2, dst2):
    n = g_lo.shape[0]
    K = KW
    nrows = src2.shape[0]
    ptr = nrows // NTILES
    IT = 10
    nit = ptr // IT
    tail = ptr % IT
    npad = _npad(n)
    ptn = npad // NTILES
    nz = ptn // ZR

    @functools.partial(
        pl.kernel, mesh=_mesh(),
        compiler_params=pltpu.CompilerParams(use_tc_tiling_on_sc=False),
        out_type=jax.ShapeDtypeStruct((NCORES, npad, 32), jnp.float32),
        scratch_types=[
            pltpu.VMEM((IT, KW), jnp.int32), pltpu.VMEM((IT, KW), jnp.int32),
            pltpu.VMEM((KW, 32), jnp.float32), pltpu.VMEM((KW, 32), jnp.float32),
            pltpu.VMEM((KW, 16), jnp.float32), pltpu.VMEM((KW, 16), jnp.float32),
            pltpu.VMEM((KW, 16), jnp.float32), pltpu.VMEM((KW, 16), jnp.float32),
            pltpu.VMEM_SHARED((npad, 32), jnp.float32),
            pltpu.SemaphoreType.DMA, pltpu.SemaphoreType.DMA,
        ],
    )
    def k(glo_h, ghi_h, dinv_h, ex_h, src_h, dst_h, zz_h, out_h,
          sib, dib, growA, growB, dvbA, dvbB, exbA, exbB, acc, semA, semB):
        core = lax.axis_index("c")
        sid = lax.axis_index("s")
        i0 = jnp.full((16,), core * 2, jnp.int32)
        i1 = jnp.full((16,), core * 2 + 1, jnp.int32)
        r0 = sid * ptn
        pltpu.sync_copy(zz_h.at[pl.ds(r0, ptn)], acc.at[pl.ds(r0, ptn)])
        plsc.subcore_barrier()
        wrow0 = sid * ptr

        def issue(j, wr, grow, dvb, exb, sem):
            @pl.when(core == 0)
            def _():
                pltpu.async_copy(glo_h.at[sib.at[j]], grow, sem)

            @pl.when(core == 1)
            def _():
                pltpu.async_copy(ghi_h.at[sib.at[j]], grow, sem)

            pltpu.async_copy(dinv_h.at[dib.at[j]], dvb, sem)
            pltpu.async_copy(ex_h.at[pl.ds((wr + j) * K, K)], exb, sem)

        def wait(grow, dvb, exb, sem):
            pltpu.make_async_copy(glo_h.at[pl.ds(0, K)], grow, sem).wait()
            pltpu.make_async_copy(dinv_h.at[pl.ds(0, K)], dvb, sem).wait()
            pltpu.make_async_copy(ex_h.at[pl.ds(0, K)], exb, sem).wait()

        def compute(grow, dvb, exb):
            def body(j, carry):
                wv = exb[j, :] * dvb[j, :]
                w0 = _lane_gather(wv, i0)
                w1 = _lane_gather(wv, i1)
                grow[j, pl.ds(0, 16)] = grow[j, pl.ds(0, 16)] * w0
                grow[j, pl.ds(16, 16)] = grow[j, pl.ds(16, 16)] * w1
                return carry

            lax.fori_loop(0, K, body, 0)

        bufs = ((growA, dvbA, exbA, semA), (growB, dvbB, exbB, semB))

        def run_group(nw, wr):
            issue(0, wr, bufs[0][0], bufs[0][1], bufs[0][2], bufs[0][3])
            for j in range(nw):
                grow, dvb, exb, sem = bufs[j % 2]
                wait(grow, dvb, exb, sem)
                if j + 1 < nw:
                    nxt = bufs[(j + 1) % 2]
                    issue(j + 1, wr, nxt[0], nxt[1], nxt[2], nxt[3])
                compute(grow, dvb, exb)
                pltpu.sync_copy(grow, acc.at[dib.at[j]], add=True)

        def it(i, carry):
            wr = wrow0 + IT * i
            pltpu.sync_copy(src_h.at[pl.ds(wr, IT)], sib)
            pltpu.sync_copy(dst_h.at[pl.ds(wr, IT)], dib)
            run_group(IT, wr)
            return carry

        lax.fori_loop(0, nit, it, 0)
        if tail:
            wr = wrow0 + IT * nit
            pltpu.sync_copy(src_h.at[pl.ds(wr, tail)], sib.at[pl.ds(0, tail)])
            pltpu.sync_copy(dst_h.at[pl.ds(wr, tail)], dib.at[pl.ds(0, tail)])
            run_group(tail, wr)
        plsc.subcore_barrier()
        pltpu.sync_copy(acc.at[pl.ds(r0, ptn)], out_h.at[core, pl.ds(r0, ptn)])

    return k(g_lo, g_hi, dinv16, ex16, src2, dst2,
             jnp.zeros((npad, 32), jnp.float32))


def _full(a):
    return pl.BlockSpec(a.shape, lambda i: tuple(0 for _ in a.shape))


def _row(width):
    return pl.BlockSpec((BN, width), lambda i: (i, 0))


def _prep_body(x_ref, nt_ref, wkw, wkb, gw, As, Ad,
               h_ref, hlo_ref, hhi_ref, glo_ref, ghi_ref, as_ref, ad_ref):
    m = (nt_ref[...] == 0).astype(jnp.float32)
    h = (x_ref[...] @ wkw[...] + wkb[...]) * m
    h_ref[...] = h
    hlo_ref[...] = h[:, :32]
    hhi_ref[...] = h[:, 32:]
    g = h @ gw[...]
    glo_ref[...] = g[:, :32]
    ghi_ref[...] = g[:, 32:]
    zs = jnp.zeros((h.shape[0], 12), jnp.float32)
    as_ref[...] = jnp.concatenate([g @ As[...], zs], axis=1)
    ad_ref[...] = jnp.concatenate([g @ Ad[...], zs], axis=1)


def _tc_prep(x, nt2, wkw, wkb, gw, As, Ad):
    n = x.shape[0]
    f = jax.ShapeDtypeStruct
    args = (x, nt2, wkw, wkb, gw, As, Ad)
    return pl.pallas_call(
        _prep_body,
        grid=(n // BN,),
        in_specs=[_row(x.shape[1]), _row(1), _full(wkw), _full(wkb),
                  _full(gw), _full(As), _full(Ad)],
        out_specs=[_row(64), _row(32), _row(32), _row(32), _row(32),
                   _row(16), _row(16)],
        out_shape=[f((n, 64), jnp.float32), f((n, 32), jnp.float32),
                   f((n, 32), jnp.float32), f((n, 32), jnp.float32),
                   f((n, 32), jnp.float32), f((n, 16), jnp.float32),
                   f((n, 16), jnp.float32)],
    )(*args)


def _mid_body(h_ref, a0_ref, a1_ref, d0_ref, d1_ref, slw, slb, srw,
              sage_ref, dinv_ref):
    dd = d0_ref[...] + d1_ref[...]
    deg = jnp.clip(dd[:, 4:5], 1.0, None)
    dinv_ref[...] = 1.0 / jnp.clip(dd, 1e-16, None)
    agg = jnp.concatenate([a0_ref[...], a1_ref[...]], axis=-1) / deg
    sage_ref[...] = jnp.maximum(
        agg @ slw[...] + slb[...] + h_ref[...] @ srw[...], 0.0)


def _tc_mid(h, a0, a1, d0, d1, slw, slb, srw):
    n = h.shape[0]
    f = jax.ShapeDtypeStruct
    return pl.pallas_call(
        _mid_body,
        grid=(n // BN,),
        in_specs=[_row(64), _row(32), _row(32), _row(16), _row(16),
                  _full(slw), _full(slb), _full(srw)],
        out_specs=[_row(64), _row(16)],
        out_shape=[f((n, 64), jnp.float32), f((n, 16), jnp.float32)],
    )(h, a0, a1, d0, d1, slw, slb, srw)


def _post_ln(sage, att0, att1, h, gatb):
    att = jnp.concatenate([att0, att1], axis=-1) + gatb
    hh = sage + att + h
    mu = jnp.mean(hh, -1, keepdims=True)
    var = jnp.mean((hh - mu) ** 2, -1, keepdims=True)
    return (hh - mu) * lax.rsqrt(var + 1e-5)


def _post0_body(sage_ref, att0_ref, att1_ref, h_ref, gatb, gw, As, Ad,
                h_ref_o, hlo_ref, hhi_ref, glo_ref, ghi_ref, as_ref, ad_ref):
    hn = _post_ln(sage_ref[...], att0_ref[...], att1_ref[...], h_ref[...],
                  gatb[...])
    h_ref_o[...] = hn
    hlo_ref[...] = hn[:, :32]
    hhi_ref[...] = hn[:, 32:]
    g = hn @ gw[...]
    glo_ref[...] = g[:, :32]
    ghi_ref[...] = g[:, 32:]
    zs = jnp.zeros((hn.shape[0], 12), jnp.float32)
    as_ref[...] = jnp.concatenate([g @ As[...], zs], axis=1)
    ad_ref[...] = jnp.concatenate([g @ Ad[...], zs], axis=1)


def _tc_post0(sage, att0, att1, h, gatb, gw, As, Ad):
    n = h.shape[0]
    f = jax.ShapeDtypeStruct
    return pl.pallas_call(
        _post0_body,
        grid=(n // BN,),
        in_specs=[_row(64), _row(32), _row(32), _row(64),
                  _full(gatb), _full(gw), _full(As), _full(Ad)],
        out_specs=[_row(64), _row(32), _row(32), _row(32), _row(32),
                   _row(16), _row(16)],
        out_shape=[f((n, 64), jnp.float32), f((n, 32), jnp.float32),
                   f((n, 32), jnp.float32), f((n, 32), jnp.float32),
                   f((n, 32), jnp.float32), f((n, 16), jnp.float32),
                   f((n, 16), jnp.float32)],
    )(sage, att0, att1, h, gatb, gw, As, Ad)


def _post1_body(sage_ref, att0_ref, att1_ref, h_ref, gatb, we_ref):
    we_ref[...] = _post_ln(sage_ref[...], att0_ref[...], att1_ref[...],
                           h_ref[...], gatb[...])


def _tc_post1(sage, att0, att1, h, gatb):
    n = h.shape[0]
    return pl.pallas_call(
        _post1_body,
        grid=(n // BN,),
        in_specs=[_row(64), _row(32), _row(32), _row(64), _full(gatb)],
        out_specs=_row(64),
        out_shape=jax.ShapeDtypeStruct((n, 64), jnp.float32),
    )(sage, att0, att1, h, gatb)


def _heads_body(we_ref, r1w, r1b, r2w, r2b, r3w, r3b, c1w, c1b, c2w, c2b,
                e1w, e1b, e2w, e2b, out_ref):
    we = we_ref[...]
    r = jnp.maximum(we @ r1w[...] + r1b[...], 0.0)
    r = jnp.maximum(r @ r2w[...] + r2b[...], 0.0)
    res = jax.nn.sigmoid(r @ r3w[...] + r3b[...])
    c = jnp.maximum(we @ c1w[...] + c1b[...], 0.0)
    conf = jax.nn.sigmoid(c @ c2w[...] + c2b[...])
    ev = jnp.maximum(we @ e1w[...] + e1b[...], 0.0)
    z = ev @ e2w[...] + e2b[...]
    z = z - jnp.max(z, axis=-1, keepdims=True)
    pz = jnp.exp(z)
    pz = pz / jnp.sum(pz, axis=-1, keepdims=True)
    out_ref[:, 0:1] = res
    out_ref[:, 1:2] = conf
    out_ref[:, 2:7] = pz
    out_ref[:, 7:8] = res


def _heads(we, p):
    n = we.shape[0]
    args = [
        p['r1_W'], p['r1_b'].reshape(1, -1),
        p['r2_W'], p['r2_b'].reshape(1, -1),
        p['r3_W'], p['r3_b'].reshape(1, -1),
        p['c1_W'], p['c1_b'].reshape(1, -1),
        p['c2_W'], p['c2_b'].reshape(1, -1),
        p['e1_W'], p['e1_b'].reshape(1, -1),
        p['e2_W'], p['e2_b'].reshape(1, -1),
    ]
    in_specs = [_row(H)] + [_full(a) for a in args]
    out = pl.pallas_call(
        _heads_body,
        grid=(n // BN,),
        in_specs=in_specs,
        out_specs=_row(8),
        out_shape=jax.ShapeDtypeStruct((n, 8), jnp.float32),
    )(we, *args)
    return out[:, 0], out[:, 1], out[:, 2:7]


def kernel(x, edge_index, edge_attr, node_type, edge_type, params):
    p = params
    n = x.shape[0]
    e = edge_index.shape[1]
    src = edge_index[0].astype(jnp.int32).reshape(e // KW, KW)
    dst = edge_index[1].astype(jnp.int32).reshape(e // KW, KW)

    def block_diag_a(a):
        return (jnp.eye(HEADS, dtype=jnp.float32)[:, None, :]
                * a[:, :, None]).reshape(H, HEADS)

    As = [block_diag_a(p[f'gat_as_{i}']) for i in range(L)]
    Ad = [block_diag_a(p[f'gat_ad_{i}']) for i in range(L)]

    h, h_lo, h_hi, g_lo, g_hi, as16, ad16 = _tc_prep(
        x, node_type.reshape(n, 1), p['wk_W'], p['wk_b'].reshape(1, -1),
        p['gat_W_0'], As[0], Ad[0])

    we = None
    for i in range(L):
        aggh = _sc_agg(h_lo, h_hi, src, dst)
        ex16, ddp = _sc_attstats(as16, ad16, src, dst)
        sage, dinv16 = _tc_mid(h, aggh[0], aggh[1], ddp[0], ddp[1],
                               p[f'sage_l_W_{i}'],
                               p[f'sage_l_b_{i}'].reshape(1, -1),
                               p[f'sage_r_W_{i}'])
        atth = _sc_att(g_lo, g_hi, dinv16, ex16, src, dst)
        gatb = p[f'gat_b_{i}'].reshape(1, -1)
        if i == 0:
            h, h_lo, h_hi, g_lo, g_hi, as16, ad16 = _tc_post0(
                sage, atth[0], atth[1], h, gatb, p['gat_W_1'], As[1], Ad[1])
        else:
            we = _tc_post1(sage, atth[0], atth[1], h, gatb)

    resilience, confidence, explanation = _heads(we, p)
    return resilience, confidence, explanation, we


# speedup vs baseline: 69.6139x; 1.0730x over previous
import functools

import jax
import jax.numpy as jnp
from jax import lax
from jax.experimental import pallas as pl
from jax.experimental.pallas import tpu as pltpu
from jax.experimental.pallas import tpu_sc as plsc

H = 64
HEADS = 4
HD = 16
L = 2
NCORES = 2
NTILES = 16
BN = 2000
KW = 200
ZR = 112


def _mesh():
    return plsc.VectorSubcoreMesh(core_axis_name="c", subcore_axis_name="s")


def _npad(n):
    return -(-n // (NTILES * ZR)) * (NTILES * ZR)


def _sc_agg(h_lo, h_hi, src2, dst2):
    n = h_lo.shape[0]
    K = src2.shape[1]
    nrows = src2.shape[0]
    ptr = nrows // NTILES
    IT = 5
    nit = ptr // IT
    tail = ptr % IT
    npad = _npad(n)
    ptn = npad // NTILES
    nz = ptn // ZR

    @functools.partial(
        pl.kernel, mesh=_mesh(),
        compiler_params=pltpu.CompilerParams(use_tc_tiling_on_sc=False),
        out_type=jax.ShapeDtypeStruct((NCORES, npad, 32), jnp.float32),
        scratch_types=[
            pltpu.VMEM((IT, K), jnp.int32), pltpu.VMEM((IT, K), jnp.int32),
            pltpu.VMEM((K, 32), jnp.float32), pltpu.VMEM((K, 32), jnp.float32),
            pltpu.VMEM_SHARED((npad, 32), jnp.float32),
            pltpu.SemaphoreType.DMA, pltpu.SemaphoreType.DMA,
        ],
    )
    def k(hlo_h, hhi_h, src_h, dst_h, zz_h, out_h,
          sib, dib, rowsA, rowsB, acc, semA, semB):
        core = lax.axis_index("c")
        sid = lax.axis_index("s")
        r0 = sid * ptn
        pltpu.sync_copy(zz_h.at[pl.ds(r0, ptn)], acc.at[pl.ds(r0, ptn)])
        plsc.subcore_barrier()
        wrow0 = sid * ptr

        def issue(j, rows, sem):
            @pl.when(core == 0)
            def _():
                pltpu.async_copy(hlo_h.at[sib.at[j]], rows, sem)

            @pl.when(core == 1)
            def _():
                pltpu.async_copy(hhi_h.at[sib.at[j]], rows, sem)

        def wait(rows, sem):
            pltpu.make_async_copy(hlo_h.at[pl.ds(0, K)], rows, sem).wait()

        bufs = ((rowsA, semA), (rowsB, semB))

        def run_group(nw):
            issue(0, *bufs[0])
            for j in range(nw):
                rows, sem = bufs[j % 2]
                wait(rows, sem)
                if j + 1 < nw:
                    issue(j + 1, *bufs[(j + 1) % 2])
                pltpu.sync_copy(rows, acc.at[dib.at[j]], add=True)

        def it(i, carry):
            wr = wrow0 + IT * i
            pltpu.sync_copy(src_h.at[pl.ds(wr, IT)], sib)
            pltpu.sync_copy(dst_h.at[pl.ds(wr, IT)], dib)
            run_group(IT)
            return carry

        lax.fori_loop(0, nit, it, 0)
        if tail:
            wr = wrow0 + IT * nit
            pltpu.sync_copy(src_h.at[pl.ds(wr, tail)], sib.at[pl.ds(0, tail)])
            pltpu.sync_copy(dst_h.at[pl.ds(wr, tail)], dib.at[pl.ds(0, tail)])
            run_group(tail)
        plsc.subcore_barrier()
        pltpu.sync_copy(acc.at[pl.ds(r0, ptn)], out_h.at[core, pl.ds(r0, ptn)])

    return k(h_lo, h_hi, src2, dst2, jnp.zeros((npad, 32), jnp.float32))


def _sc_attstats(as16, ad16, src2, dst2):
    n = as16.shape[0]
    K = KW
    nrows = src2.shape[0]
    e = nrows * K
    hr = nrows // NCORES
    ptr = hr // NTILES
    IT = 10
    nit = ptr // IT
    tail = ptr % IT
    npad = _npad(n)
    ptn = npad // NTILES
    nz = ptn // ZR

    @functools.partial(
        pl.kernel, mesh=_mesh(),
        compiler_params=pltpu.CompilerParams(use_tc_tiling_on_sc=False),
        out_type=[
            jax.ShapeDtypeStruct((e, 16), jnp.float32),
            jax.ShapeDtypeStruct((NCORES, npad, 16), jnp.float32),
        ],
        scratch_types=[
            pltpu.VMEM((IT, KW), jnp.int32), pltpu.VMEM((IT, KW), jnp.int32),
            pltpu.VMEM((KW, 16), jnp.float32), pltpu.VMEM((KW, 16), jnp.float32),
            pltpu.VMEM((KW, 16), jnp.float32), pltpu.VMEM((KW, 16), jnp.float32),
            pltpu.VMEM((KW, 16), jnp.float32), pltpu.VMEM((KW, 16), jnp.float32),
            pltpu.VMEM_SHARED((npad, 16), jnp.float32),
            pltpu.SemaphoreType.DMA, pltpu.SemaphoreType.DMA,
        ],
    )
    def k(as_h, ad_h, src_h, dst_h, zz_h, ex_h, ddp_h,
          sib, dib, asbA, adbA, exbA, asbB, adbB, exbB, acc, semA, semB):
        core = lax.axis_index("c")
        sid = lax.axis_index("s")
        r0 = sid * ptn
        pltpu.sync_copy(zz_h.at[pl.ds(r0, ptn)], acc.at[pl.ds(r0, ptn)])
        plsc.subcore_barrier()
        wrow0 = core * hr + sid * ptr

        def issue(j, asb, adb, sem):
            pltpu.async_copy(as_h.at[sib.at[j]], asb, sem)
            pltpu.async_copy(ad_h.at[dib.at[j]], adb, sem)

        def wait(asb, adb, sem):
            pltpu.make_async_copy(as_h.at[pl.ds(0, K)], asb, sem).wait()
            pltpu.make_async_copy(ad_h.at[pl.ds(0, K)], adb, sem).wait()

        def compute(asb, adb, exb):
            def body(j, carry):
                xv = asb[j, :] + adb[j, :]
                exb[j, :] = jnp.exp(jnp.maximum(xv, 0.2 * xv))
                return carry

            lax.fori_loop(0, K, body, 0)

        bufs = ((asbA, adbA, exbA, semA), (asbB, adbB, exbB, semB))

        def run_group(nw, wr):
            issue(0, bufs[0][0], bufs[0][1], bufs[0][3])
            for j in range(nw):
                asb, adb, exb, sem = bufs[j % 2]
                wait(asb, adb, sem)
                if j + 1 < nw:
                    nxt = bufs[(j + 1) % 2]
                    issue(j + 1, nxt[0], nxt[1], nxt[3])
                compute(asb, adb, exb)
                pltpu.sync_copy(exb, acc.at[dib.at[j]], add=True)
                pltpu.sync_copy(exb, ex_h.at[pl.ds((wr + j) * K, K)])

        def it(i, carry):
            wr = wrow0 + IT * i
            pltpu.sync_copy(src_h.at[pl.ds(wr, IT)], sib)
            pltpu.sync_copy(dst_h.at[pl.ds(wr, IT)], dib)
            run_group(IT, wr)
            return carry

        lax.fori_loop(0, nit, it, 0)
        if tail:
            wr = wrow0 + IT * nit
            pltpu.sync_copy(src_h.at[pl.ds(wr, tail)], sib.at[pl.ds(0, tail)])
            pltpu.sync_copy(dst_h.at[pl.ds(wr, tail)], dib.at[pl.ds(0, tail)])
            run_group(tail, wr)
        plsc.subcore_barrier()
        pltpu.sync_copy(acc.at[pl.ds(r0, ptn)], ddp_h.at[core, pl.ds(r0, ptn)])

    return k(as16, ad16, src2, dst2, jnp.zeros((npad, 16), jnp.float32))


_GD = lax.GatherDimensionNumbers(offset_dims=(), collapsed_slice_dims=(0,),
                                 start_index_map=(0,))


def _lane_gather(v, idx):
    return lax.gather(v, idx[:, None], _GD, (1,),
                      mode=lax.GatherScatterMode.PROMISE_IN_BOUNDS)


def _sc_att(g_lo, g_hi, dinv16, ex16, src2, dst2):
    n = g_lo.shape[0]
    K = KW
    nrows = src2.shape[0]
    ptr = nrows // NTILES
    IT = 10
    nit = ptr // IT
    tail = ptr % IT
    npad = _npad(n)
    ptn = npad // NTILES
    nz = ptn // ZR

    @functools.partial(
        pl.kernel, mesh=_mesh(),
        compiler_params=pltpu.CompilerParams(use_tc_tiling_on_sc=False),
        out_type=jax.ShapeDtypeStruct((NCORES, npad, 32), jnp.float32),
        scratch_types=[
            pltpu.VMEM((IT, KW), jnp.int32), pltpu.VMEM((IT, KW), jnp.int32),
            pltpu.VMEM((KW, 32), jnp.float32), pltpu.VMEM((KW, 32), jnp.float32),
            pltpu.VMEM((KW, 16), jnp.float32), pltpu.VMEM((KW, 16), jnp.float32),
            pltpu.VMEM((KW, 16), jnp.float32), pltpu.VMEM((KW, 16), jnp.float32),
            pltpu.VMEM_SHARED((npad, 32), jnp.float32),
            pltpu.SemaphoreType.DMA, pltpu.SemaphoreType.DMA,
        ],
    )
    def k(glo_h, ghi_h, dinv_h, ex_h, src_h, dst_h, zz_h, out_h,
          sib, dib, growA, growB, dvbA, dvbB, exbA, exbB, acc, semA, semB):
        core = lax.axis_index("c")
        sid = lax.axis_index("s")
        i0 = jnp.full((16,), core * 2, jnp.int32)
        i1 = jnp.full((16,), core * 2 + 1, jnp.int32)
        r0 = sid * ptn
        pltpu.sync_copy(zz_h.at[pl.ds(r0, ptn)], acc.at[pl.ds(r0, ptn)])
        plsc.subcore_barrier()
        wrow0 = sid * ptr

        def issue(j, wr, grow, dvb, exb, sem):
            @pl.when(core == 0)
            def _():
                pltpu.async_copy(glo_h.at[sib.at[j]], grow, sem)

            @pl.when(core == 1)
            def _():
                pltpu.async_copy(ghi_h.at[sib.at[j]], grow, sem)

            pltpu.async_copy(dinv_h.at[dib.at[j]], dvb, sem)
            pltpu.async_copy(ex_h.at[pl.ds((wr + j) * K, K)], exb, sem)

        def wait(grow, dvb, exb, sem):
            pltpu.make_async_copy(glo_h.at[pl.ds(0, K)], grow, sem).wait()
            pltpu.make_async_copy(dinv_h.at[pl.ds(0, K)], dvb, sem).wait()
            pltpu.make_async_copy(ex_h.at[pl.ds(0, K)], exb, sem).wait()

        def compute(grow, dvb, exb):
            def body(j, carry):
                wv = exb[j, :] * dvb[j, :]
                w0 = _lane_gather(wv, i0)
                w1 = _lane_gather(wv, i1)
                grow[j, pl.ds(0, 16)] = grow[j, pl.ds(0, 16)] * w0
                grow[j, pl.ds(16, 16)] = grow[j, pl.ds(16, 16)] * w1
                return carry

            lax.fori_loop(0, K, body, 0)

        bufs = ((growA, dvbA, exbA, semA), (growB, dvbB, exbB, semB))

        def run_group(nw, wr):
            issue(0, wr, bufs[0][0], bufs[0][1], bufs[0][2], bufs[0][3])
            for j in range(nw):
                grow, dvb, exb, sem = bufs[j % 2]
                wait(grow, dvb, exb, sem)
                if j + 1 < nw:
                    nxt = bufs[(j + 1) % 2]
                    issue(j + 1, wr, nxt[0], nxt[1], nxt[2], nxt[3])
                compute(grow, dvb, exb)
                pltpu.sync_copy(grow, acc.at[dib.at[j]], add=True)

        def it(i, carry):
            wr = wrow0 + IT * i
            pltpu.sync_copy(src_h.at[pl.ds(wr, IT)], sib)
            pltpu.sync_copy(dst_h.at[pl.ds(wr, IT)], dib)
            run_group(IT, wr)
            return carry

        lax.fori_loop(0, nit, it, 0)
        if tail:
            wr = wrow0 + IT * nit
            pltpu.sync_copy(src_h.at[pl.ds(wr, tail)], sib.at[pl.ds(0, tail)])
            pltpu.sync_copy(dst_h.at[pl.ds(wr, tail)], dib.at[pl.ds(0, tail)])
            run_group(tail, wr)
        plsc.subcore_barrier()
        pltpu.sync_copy(acc.at[pl.ds(r0, ptn)], out_h.at[core, pl.ds(r0, ptn)])

    return k(g_lo, g_hi, dinv16, ex16, src2, dst2,
             jnp.zeros((npad, 32), jnp.float32))


def _full(a):
    return pl.BlockSpec(a.shape, lambda i: tuple(0 for _ in a.shape))


def _row(width):
    return pl.BlockSpec((BN, width), lambda i: (i, 0))


def _prep_body(x_ref, nt_ref, wkw, wkb, gw, As, Ad,
               h_ref, hlo_ref, hhi_ref, glo_ref, ghi_ref, as_ref, ad_ref):
    m = (nt_ref[...] == 0).astype(jnp.float32)
    h = (x_ref[...] @ wkw[...] + wkb[...]) * m
    h_ref[...] = h
    hlo_ref[...] = h[:, :32]
    hhi_ref[...] = h[:, 32:]
    g = h @ gw[...]
    glo_ref[...] = g[:, :32]
    ghi_ref[...] = g[:, 32:]
    zs = jnp.zeros((h.shape[0], 12), jnp.float32)
    as_ref[...] = jnp.concatenate([g @ As[...], zs], axis=1)
    ad_ref[...] = jnp.concatenate([g @ Ad[...], zs], axis=1)


def _tc_prep(x, nt2, wkw, wkb, gw, As, Ad):
    n = x.shape[0]
    f = jax.ShapeDtypeStruct
    args = (x, nt2, wkw, wkb, gw, As, Ad)
    return pl.pallas_call(
        _prep_body,
        grid=(n // BN,),
        in_specs=[_row(x.shape[1]), _row(1), _full(wkw), _full(wkb),
                  _full(gw), _full(As), _full(Ad)],
        out_specs=[_row(64), _row(32), _row(32), _row(32), _row(32),
                   _row(16), _row(16)],
        out_shape=[f((n, 64), jnp.float32), f((n, 32), jnp.float32),
                   f((n, 32), jnp.float32), f((n, 32), jnp.float32),
                   f((n, 32), jnp.float32), f((n, 16), jnp.float32),
                   f((n, 16), jnp.float32)],
    )(*args)


def _mid_body(h_ref, a0_ref, a1_ref, d0_ref, d1_ref, slw, slb, srw,
              sage_ref, dinv_ref):
    dd = d0_ref[...] + d1_ref[...]
    deg = jnp.clip(dd[:, 4:5], 1.0, None)
    dinv_ref[...] = 1.0 / jnp.clip(dd, 1e-16, None)
    agg = jnp.concatenate([a0_ref[...], a1_ref[...]], axis=-1) / deg
    sage_ref[...] = jnp.maximum(
        agg @ slw[...] + slb[...] + h_ref[...] @ srw[...], 0.0)


def _tc_mid(h, a0, a1, d0, d1, slw, slb, srw):
    n = h.shape[0]
    f = jax.ShapeDtypeStruct
    return pl.pallas_call(
        _mid_body,
        grid=(n // BN,),
        in_specs=[_row(64), _row(32), _row(32), _row(16), _row(16),
                  _full(slw), _full(slb), _full(srw)],
        out_specs=[_row(64), _row(16)],
        out_shape=[f((n, 64), jnp.float32), f((n, 16), jnp.float32)],
    )(h, a0, a1, d0, d1, slw, slb, srw)


def _post_ln(sage, att0, att1, h, gatb):
    att = jnp.concatenate([att0, att1], axis=-1) + gatb
    hh = sage + att + h
    mu = jnp.mean(hh, -1, keepdims=True)
    var = jnp.mean((hh - mu) ** 2, -1, keepdims=True)
    return (hh - mu) * lax.rsqrt(var + 1e-5)


def _post0_body(sage_ref, att0_ref, att1_ref, h_ref, gatb, gw, As, Ad,
                h_ref_o, hlo_ref, hhi_ref, glo_ref, ghi_ref, as_ref, ad_ref):
    hn = _post_ln(sage_ref[...], att0_ref[...], att1_ref[...], h_ref[...],
                  gatb[...])
    h_ref_o[...] = hn
    hlo_ref[...] = hn[:, :32]
    hhi_ref[...] = hn[:, 32:]
    g = hn @ gw[...]
    glo_ref[...] = g[:, :32]
    ghi_ref[...] = g[:, 32:]
    zs = jnp.zeros((hn.shape[0], 12), jnp.float32)
    as_ref[...] = jnp.concatenate([g @ As[...], zs], axis=1)
    ad_ref[...] = jnp.concatenate([g @ Ad[...], zs], axis=1)


def _tc_post0(sage, att0, att1, h, gatb, gw, As, Ad):
    n = h.shape[0]
    f = jax.ShapeDtypeStruct
    return pl.pallas_call(
        _post0_body,
        grid=(n // BN,),
        in_specs=[_row(64), _row(32), _row(32), _row(64),
                  _full(gatb), _full(gw), _full(As), _full(Ad)],
        out_specs=[_row(64), _row(32), _row(32), _row(32), _row(32),
                   _row(16), _row(16)],
        out_shape=[f((n, 64), jnp.float32), f((n, 32), jnp.float32),
                   f((n, 32), jnp.float32), f((n, 32), jnp.float32),
                   f((n, 32), jnp.float32), f((n, 16), jnp.float32),
                   f((n, 16), jnp.float32)],
    )(sage, att0, att1, h, gatb, gw, As, Ad)


def _post1_body(sage_ref, att0_ref, att1_ref, h_ref, gatb,
                r1w, r1b, r2w, r2b, r3w, r3b, c1w, c1b, c2w, c2b,
                e1w, e1b, e2w, e2b, we_ref, out_ref):
    we = _post_ln(sage_ref[...], att0_ref[...], att1_ref[...], h_ref[...],
                  gatb[...])
    we_ref[...] = we
    r = jnp.maximum(we @ r1w[...] + r1b[...], 0.0)
    r = jnp.maximum(r @ r2w[...] + r2b[...], 0.0)
    res = jax.nn.sigmoid(r @ r3w[...] + r3b[...])
    c = jnp.maximum(we @ c1w[...] + c1b[...], 0.0)
    conf = jax.nn.sigmoid(c @ c2w[...] + c2b[...])
    ev = jnp.maximum(we @ e1w[...] + e1b[...], 0.0)
    z = ev @ e2w[...] + e2b[...]
    z = z - jnp.max(z, axis=-1, keepdims=True)
    pz = jnp.exp(z)
    pz = pz / jnp.sum(pz, axis=-1, keepdims=True)
    out_ref[:, 0:1] = res
    out_ref[:, 1:2] = conf
    out_ref[:, 2:7] = pz
    out_ref[:, 7:8] = res


def _tc_post1(sage, att0, att1, h, gatb, p):
    n = h.shape[0]
    hargs = [
        p['r1_W'], p['r1_b'].reshape(1, -1),
        p['r2_W'], p['r2_b'].reshape(1, -1),
        p['r3_W'], p['r3_b'].reshape(1, -1),
        p['c1_W'], p['c1_b'].reshape(1, -1),
        p['c2_W'], p['c2_b'].reshape(1, -1),
        p['e1_W'], p['e1_b'].reshape(1, -1),
        p['e2_W'], p['e2_b'].reshape(1, -1),
    ]
    f = jax.ShapeDtypeStruct
    return pl.pallas_call(
        _post1_body,
        grid=(n // BN,),
        in_specs=[_row(64), _row(32), _row(32), _row(64), _full(gatb)]
        + [_full(a) for a in hargs],
        out_specs=[_row(64), _row(8)],
        out_shape=[f((n, 64), jnp.float32), f((n, 8), jnp.float32)],
    )(sage, att0, att1, h, gatb, *hargs)


def _heads_body(we_ref, r1w, r1b, r2w, r2b, r3w, r3b, c1w, c1b, c2w, c2b,
                e1w, e1b, e2w, e2b, out_ref):
    we = we_ref[...]
    r = jnp.maximum(we @ r1w[...] + r1b[...], 0.0)
    r = jnp.maximum(r @ r2w[...] + r2b[...], 0.0)
    res = jax.nn.sigmoid(r @ r3w[...] + r3b[...])
    c = jnp.maximum(we @ c1w[...] + c1b[...], 0.0)
    conf = jax.nn.sigmoid(c @ c2w[...] + c2b[...])
    ev = jnp.maximum(we @ e1w[...] + e1b[...], 0.0)
    z = ev @ e2w[...] + e2b[...]
    z = z - jnp.max(z, axis=-1, keepdims=True)
    pz = jnp.exp(z)
    pz = pz / jnp.sum(pz, axis=-1, keepdims=True)
    out_ref[:, 0:1] = res
    out_ref[:, 1:2] = conf
    out_ref[:, 2:7] = pz
    out_ref[:, 7:8] = res


def _heads(we, p):
    n = we.shape[0]
    args = [
        p['r1_W'], p['r1_b'].reshape(1, -1),
        p['r2_W'], p['r2_b'].reshape(1, -1),
        p['r3_W'], p['r3_b'].reshape(1, -1),
        p['c1_W'], p['c1_b'].reshape(1, -1),
        p['c2_W'], p['c2_b'].reshape(1, -1),
        p['e1_W'], p['e1_b'].reshape(1, -1),
        p['e2_W'], p['e2_b'].reshape(1, -1),
    ]
    in_specs = [_row(H)] + [_full(a) for a in args]
    out = pl.pallas_call(
        _heads_body,
        grid=(n // BN,),
        in_specs=in_specs,
        out_specs=_row(8),
        out_shape=jax.ShapeDtypeStruct((n, 8), jnp.float32),
    )(we, *args)
    return out[:, 0], out[:, 1], out[:, 2:7]


def kernel(x, edge_index, edge_attr, node_type, edge_type, params):
    p = params
    n = x.shape[0]
    e = edge_index.shape[1]
    src = edge_index[0].astype(jnp.int32).reshape(e // KW, KW)
    dst = edge_index[1].astype(jnp.int32).reshape(e // KW, KW)
    src4 = src.reshape(e // 400, 400)
    dst4 = dst.reshape(e // 400, 400)

    def block_diag_a(a):
        return (jnp.eye(HEADS, dtype=jnp.float32)[:, None, :]
                * a[:, :, None]).reshape(H, HEADS)

    As = [block_diag_a(p[f'gat_as_{i}']) for i in range(L)]
    Ad = [block_diag_a(p[f'gat_ad_{i}']) for i in range(L)]

    h, h_lo, h_hi, g_lo, g_hi, as16, ad16 = _tc_prep(
        x, node_type.reshape(n, 1), p['wk_W'], p['wk_b'].reshape(1, -1),
        p['gat_W_0'], As[0], Ad[0])

    we = None
    for i in range(L):
        aggh = _sc_agg(h_lo, h_hi, src4, dst4)
        ex16, ddp = _sc_attstats(as16, ad16, src, dst)
        sage, dinv16 = _tc_mid(h, aggh[0], aggh[1], ddp[0], ddp[1],
                               p[f'sage_l_W_{i}'],
                               p[f'sage_l_b_{i}'].reshape(1, -1),
                               p[f'sage_r_W_{i}'])
        atth = _sc_att(g_lo, g_hi, dinv16, ex16, src, dst)
        gatb = p[f'gat_b_{i}'].reshape(1, -1)
        if i == 0:
            h, h_lo, h_hi, g_lo, g_hi, as16, ad16 = _tc_post0(
                sage, atth[0], atth[1], h, gatb, p['gat_W_1'], As[1], Ad[1])
        else:
            we, ho = _tc_post1(sage, atth[0], atth[1], h, gatb, p)

    return ho[:, 0], ho[:, 1], ho[:, 2:7], we
